# Initial kernel scaffold; baseline (speedup 1.0000x reference)
import jax
import jax.numpy as jnp
from jax.experimental import pallas as pl


def kernel(A, x, trafo):
    raise NotImplementedError("write your pallas kernel here")


# speedup vs baseline: 1.6635x; 1.6635x over previous
import jax
import jax.numpy as jnp
from jax import lax
from jax.experimental import pallas as pl

C_CONST = 1000.0
CUT = 0.5
HI = lax.Precision.HIGHEST


def _sort_key(v):
    b = lax.bitcast_convert_type(v, jnp.int32)
    return jnp.where(b >= 0, b, b ^ jnp.int32(0x7FFFFFFF))


def _rank_of_nodes(vrow, vcol, n):
    krow = _sort_key(vrow)
    kcol = _sort_key(vcol)
    i_col = lax.broadcasted_iota(jnp.int32, (n, n), 0)
    j_row = lax.broadcasted_iota(jnp.int32, (n, n), 1)
    beats = (kcol > krow) | ((kcol == krow) & (i_col < j_row))
    return jnp.sum(beats.astype(jnp.int32), axis=0, keepdims=True)


def _a_body(vrow_ref, vcol_ref, a_ref, out_ref):
    a = a_ref[0]
    n = a.shape[0]
    og = n // 2
    rank = _rank_of_nodes(vrow_ref[0], vcol_ref[0], n)
    o_col = lax.broadcasted_iota(jnp.int32, (og, n), 0)
    s = ((rank // 2) == o_col).astype(jnp.float32)
    rowsum = lax.dot_general(s, a, (((1,), (0,)), ((), ())),
                             preferred_element_type=jnp.float32, precision=HI)
    am = 0.25 * lax.dot_general(rowsum, s, (((1,), (1,)), ((), ())),
                                preferred_element_type=jnp.float32, precision=HI)
    t = C_CONST * (am - CUT)
    out_ref[0] = jnp.maximum(1.0 + t, 0.0) - jnp.maximum(t, 0.0)


def _x_body(vrow_ref, vcol_ref, x_ref, w_ref, out_ref):
    xb = x_ref[0]
    n = xb.shape[0]
    p = xb.shape[1]
    og = n // 2
    rank = _rank_of_nodes(vrow_ref[0], vcol_ref[0], n)
    o_col = lax.broadcasted_iota(jnp.int32, (og, n), 0)
    p1 = (rank == 2 * o_col).astype(jnp.float32)
    p2 = (rank == 2 * o_col + 1).astype(jnp.float32)
    xge = lax.dot_general(p1, xb, (((1,), (0,)), ((), ())),
                          preferred_element_type=jnp.float32, precision=HI)
    xgo = lax.dot_general(p2, xb, (((1,), (0,)), ((), ())),
                          preferred_element_type=jnp.float32, precision=HI)
    w1 = w_ref[:p, :]
    w2 = w_ref[p:, :]
    out_ref[0] = (
        lax.dot_general(xge, w1, (((1,), (0,)), ((), ())),
                        preferred_element_type=jnp.float32, precision=HI)
        + lax.dot_general(xgo, w2, (((1,), (0,)), ((), ())),
                          preferred_element_type=jnp.float32, precision=HI))


def kernel(A, x, trafo):
    b, n, p = x.shape
    og = n // 2
    po = trafo.shape[1]
    values = x[:, :, -1]
    vrow = values.reshape(b, 1, n)
    vcol = values.reshape(b, n, 1)

    ar = pl.pallas_call(
        _a_body,
        grid=(b,),
        in_specs=[
            pl.BlockSpec((1, 1, n), lambda i: (i, 0, 0)),
            pl.BlockSpec((1, n, 1), lambda i: (i, 0, 0)),
            pl.BlockSpec((1, n, n), lambda i: (i, 0, 0)),
        ],
        out_specs=pl.BlockSpec((1, og, og), lambda i: (i, 0, 0)),
        out_shape=jax.ShapeDtypeStruct((b, og, og), jnp.float32),
    )(vrow, vcol, A)

    traf = pl.pallas_call(
        _x_body,
        grid=(b,),
        in_specs=[
            pl.BlockSpec((1, 1, n), lambda i: (i, 0, 0)),
            pl.BlockSpec((1, n, 1), lambda i: (i, 0, 0)),
            pl.BlockSpec((1, n, p), lambda i: (i, 0, 0)),
            pl.BlockSpec((2 * p, po), lambda i: (0, 0)),
        ],
        out_specs=pl.BlockSpec((1, og, po), lambda i: (i, 0, 0)),
        out_shape=jax.ShapeDtypeStruct((b, og, po), jnp.float32),
    )(vrow, vcol, x, trafo)

    return ar, traf


# speedup vs baseline: 3.8479x; 2.3132x over previous
import jax
import jax.numpy as jnp
from jax import lax
from jax.experimental import pallas as pl

C_CONST = 1000.0
CUT = 0.5
HI = lax.Precision.HIGHEST
MED = lax.Precision.HIGH
LO = lax.Precision.DEFAULT


def _sort_key(v):
    b = lax.bitcast_convert_type(v, jnp.int32)
    return jnp.where(b >= 0, b, b ^ jnp.int32(0x7FFFFFFF))


def _rank_of_nodes(vrow, vcol, n):
    krow = _sort_key(vrow)
    kcol = _sort_key(vcol)
    i_col = lax.broadcasted_iota(jnp.int32, (n, n), 0)
    j_row = lax.broadcasted_iota(jnp.int32, (n, n), 1)
    beats = (kcol > krow) | ((kcol == krow) & (i_col < j_row))
    return jnp.sum(beats.astype(jnp.int32), axis=0, keepdims=True)


def _a_body(vrow_ref, vcol_ref, a_ref, out_ref):
    a = a_ref[0]
    n = a.shape[0]
    og = n // 2
    rank = _rank_of_nodes(vrow_ref[0], vcol_ref[0], n)
    o_col = lax.broadcasted_iota(jnp.int32, (og, n), 0)
    s = ((rank // 2) == o_col).astype(jnp.float32)
    rowsum = lax.dot_general(s, a, (((1,), (0,)), ((), ())),
                             preferred_element_type=jnp.float32, precision=HI)
    am = 0.25 * lax.dot_general(rowsum, s, (((1,), (1,)), ((), ())),
                                preferred_element_type=jnp.float32, precision=HI)
    t = C_CONST * (am - CUT)
    out_ref[0] = jnp.maximum(1.0 + t, 0.0) - jnp.maximum(t, 0.0)


def _x_body(vrow_ref, vcol_ref, x_ref, w_ref, out_ref):
    xb = x_ref[0]
    n = xb.shape[0]
    p = xb.shape[1]
    og = n // 2
    rank = _rank_of_nodes(vrow_ref[0], vcol_ref[0], n)
    o_col = lax.broadcasted_iota(jnp.int32, (og, n), 0)
    p1 = (rank == 2 * o_col).astype(jnp.float32)
    p2 = (rank == 2 * o_col + 1).astype(jnp.float32)
    xge = lax.dot_general(p1, xb, (((1,), (0,)), ((), ())),
                          preferred_element_type=jnp.float32, precision=LO)
    xgo = lax.dot_general(p2, xb, (((1,), (0,)), ((), ())),
                          preferred_element_type=jnp.float32, precision=LO)
    w1 = w_ref[:p, :]
    w2 = w_ref[p:, :]
    out_ref[0] = (
        lax.dot_general(xge, w1, (((1,), (0,)), ((), ())),
                        preferred_element_type=jnp.float32, precision=LO)
        + lax.dot_general(xgo, w2, (((1,), (0,)), ((), ())),
                          preferred_element_type=jnp.float32, precision=LO))


def kernel(A, x, trafo):
    b, n, p = x.shape
    og = n // 2
    po = trafo.shape[1]
    values = x[:, :, -1]
    vrow = values.reshape(b, 1, n)
    vcol = values.reshape(b, n, 1)

    ar = pl.pallas_call(
        _a_body,
        grid=(b,),
        in_specs=[
            pl.BlockSpec((1, 1, n), lambda i: (i, 0, 0)),
            pl.BlockSpec((1, n, 1), lambda i: (i, 0, 0)),
            pl.BlockSpec((1, n, n), lambda i: (i, 0, 0)),
        ],
        out_specs=pl.BlockSpec((1, og, og), lambda i: (i, 0, 0)),
        out_shape=jax.ShapeDtypeStruct((b, og, og), jnp.float32),
    )(vrow, vcol, A)

    traf = pl.pallas_call(
        _x_body,
        grid=(b,),
        in_specs=[
            pl.BlockSpec((1, 1, n), lambda i: (i, 0, 0)),
            pl.BlockSpec((1, n, 1), lambda i: (i, 0, 0)),
            pl.BlockSpec((1, n, p), lambda i: (i, 0, 0)),
            pl.BlockSpec((2 * p, po), lambda i: (0, 0)),
        ],
        out_specs=pl.BlockSpec((1, og, po), lambda i: (i, 0, 0)),
        out_shape=jax.ShapeDtypeStruct((b, og, po), jnp.float32),
    )(vrow, vcol, x, trafo)

    return ar, traf


# speedup vs baseline: 6.3049x; 1.6385x over previous
import jax
import jax.numpy as jnp
from jax import lax
from jax.experimental import pallas as pl

C_CONST = 1000.0
CUT = 0.5
HI = lax.Precision.HIGHEST
MED = lax.Precision.HIGH
LO = lax.Precision.DEFAULT


def _sort_key(v):
    b = lax.bitcast_convert_type(v, jnp.int32)
    return jnp.where(b >= 0, b, b ^ jnp.int32(0x7FFFFFFF))


def _rank_of_nodes(vrow, vcol, n):
    krow = _sort_key(vrow)
    kcol = _sort_key(vcol)
    i_col = lax.broadcasted_iota(jnp.int32, (n, n), 0)
    j_row = lax.broadcasted_iota(jnp.int32, (n, n), 1)
    beats = (kcol > krow) | ((kcol == krow) & (i_col < j_row))
    return jnp.sum(beats.astype(jnp.int32), axis=0, keepdims=True)


def _a_body(vrow_ref, vcol_ref, a_ref, out_ref):
    a = a_ref[0]
    n = a.shape[0]
    og = n // 2
    rank = _rank_of_nodes(vrow_ref[0], vcol_ref[0], n)
    o_col = lax.broadcasted_iota(jnp.int32, (og, n), 0)
    s = ((rank // 2) == o_col).astype(jnp.bfloat16)
    a1 = a.astype(jnp.bfloat16)
    a2 = (a - a1.astype(jnp.float32)).astype(jnp.bfloat16)
    rowsum = (lax.dot_general(s, a1, (((1,), (0,)), ((), ())),
                              preferred_element_type=jnp.float32)
              + lax.dot_general(s, a2, (((1,), (0,)), ((), ())),
                                preferred_element_type=jnp.float32))
    r1 = rowsum.astype(jnp.bfloat16)
    r2 = (rowsum - r1.astype(jnp.float32)).astype(jnp.bfloat16)
    am = 0.25 * (lax.dot_general(r1, s, (((1,), (1,)), ((), ())),
                                 preferred_element_type=jnp.float32)
                 + lax.dot_general(r2, s, (((1,), (1,)), ((), ())),
                                   preferred_element_type=jnp.float32))
    t = C_CONST * (am - CUT)
    out_ref[0] = jnp.maximum(1.0 + t, 0.0) - jnp.maximum(t, 0.0)


def _x_body(vrow_ref, vcol_ref, x_ref, w_ref, out_ref):
    xb = x_ref[0]
    n = xb.shape[0]
    p = xb.shape[1]
    og = n // 2
    rank = _rank_of_nodes(vrow_ref[0], vcol_ref[0], n)
    o_col = lax.broadcasted_iota(jnp.int32, (og, n), 0)
    p1 = (rank == 2 * o_col).astype(jnp.float32)
    p2 = (rank == 2 * o_col + 1).astype(jnp.float32)
    xge = lax.dot_general(p1, xb, (((1,), (0,)), ((), ())),
                          preferred_element_type=jnp.float32, precision=LO)
    xgo = lax.dot_general(p2, xb, (((1,), (0,)), ((), ())),
                          preferred_element_type=jnp.float32, precision=LO)
    w1 = w_ref[:p, :]
    w2 = w_ref[p:, :]
    out_ref[0] = (
        lax.dot_general(xge, w1, (((1,), (0,)), ((), ())),
                        preferred_element_type=jnp.float32, precision=LO)
        + lax.dot_general(xgo, w2, (((1,), (0,)), ((), ())),
                          preferred_element_type=jnp.float32, precision=LO))


def kernel(A, x, trafo):
    b, n, p = x.shape
    og = n // 2
    po = trafo.shape[1]
    values = x[:, :, -1]
    vrow = values.reshape(b, 1, n)
    vcol = values.reshape(b, n, 1)

    ar = pl.pallas_call(
        _a_body,
        grid=(b,),
        in_specs=[
            pl.BlockSpec((1, 1, n), lambda i: (i, 0, 0)),
            pl.BlockSpec((1, n, 1), lambda i: (i, 0, 0)),
            pl.BlockSpec((1, n, n), lambda i: (i, 0, 0)),
        ],
        out_specs=pl.BlockSpec((1, og, og), lambda i: (i, 0, 0)),
        out_shape=jax.ShapeDtypeStruct((b, og, og), jnp.float32),
    )(vrow, vcol, A)

    traf = pl.pallas_call(
        _x_body,
        grid=(b,),
        in_specs=[
            pl.BlockSpec((1, 1, n), lambda i: (i, 0, 0)),
            pl.BlockSpec((1, n, 1), lambda i: (i, 0, 0)),
            pl.BlockSpec((1, n, p), lambda i: (i, 0, 0)),
            pl.BlockSpec((2 * p, po), lambda i: (0, 0)),
        ],
        out_specs=pl.BlockSpec((1, og, po), lambda i: (i, 0, 0)),
        out_shape=jax.ShapeDtypeStruct((b, og, po), jnp.float32),
    )(vrow, vcol, x, trafo)

    return ar, traf


# speedup vs baseline: 6.9378x; 1.1004x over previous
import jax
import jax.numpy as jnp
from jax import lax
from jax.experimental import pallas as pl

C_CONST = 1000.0
CUT = 0.5
LO = lax.Precision.DEFAULT


def _sort_key(v):
    b = lax.bitcast_convert_type(v, jnp.int32)
    return jnp.where(b >= 0, b, b ^ jnp.int32(0x7FFFFFFF))


def _rank_of_nodes(vrow, vcol, n):
    krow = _sort_key(vrow)
    kcol = _sort_key(vcol)
    i_col = lax.broadcasted_iota(jnp.int32, (n, n), 0)
    j_row = lax.broadcasted_iota(jnp.int32, (n, n), 1)
    beats = (kcol > krow) | ((kcol == krow) & (i_col < j_row))
    return jnp.sum(beats.astype(jnp.int32), axis=0, keepdims=True)


def _fused_body(vrow_ref, vcol_ref, a_ref, x_ref, w_ref, ar_ref, traf_ref):
    a = a_ref[0]
    xb = x_ref[0]
    n = a.shape[0]
    p = xb.shape[1]
    og = n // 2
    rank = _rank_of_nodes(vrow_ref[0], vcol_ref[0], n)
    o_col = lax.broadcasted_iota(jnp.int32, (og, n), 0)

    p1 = (rank == 2 * o_col).astype(jnp.float32)
    p2 = (rank == 2 * o_col + 1).astype(jnp.float32)
    xge = lax.dot_general(p1, xb, (((1,), (0,)), ((), ())),
                          preferred_element_type=jnp.float32, precision=LO)
    xgo = lax.dot_general(p2, xb, (((1,), (0,)), ((), ())),
                          preferred_element_type=jnp.float32, precision=LO)
    w1 = w_ref[:p, :]
    w2 = w_ref[p:, :]
    traf_ref[0] = (
        lax.dot_general(xge, w1, (((1,), (0,)), ((), ())),
                        preferred_element_type=jnp.float32, precision=LO)
        + lax.dot_general(xgo, w2, (((1,), (0,)), ((), ())),
                          preferred_element_type=jnp.float32, precision=LO))

    s = ((rank // 2) == o_col).astype(jnp.bfloat16)
    a1 = a.astype(jnp.bfloat16)
    a2 = (a - a1.astype(jnp.float32)).astype(jnp.bfloat16)
    rowsum = (lax.dot_general(s, a1, (((1,), (0,)), ((), ())),
                              preferred_element_type=jnp.float32)
              + lax.dot_general(s, a2, (((1,), (0,)), ((), ())),
                                preferred_element_type=jnp.float32))
    r1 = rowsum.astype(jnp.bfloat16)
    r2 = (rowsum - r1.astype(jnp.float32)).astype(jnp.bfloat16)
    am = 0.25 * (lax.dot_general(r1, s, (((1,), (1,)), ((), ())),
                                 preferred_element_type=jnp.float32)
                 + lax.dot_general(r2, s, (((1,), (1,)), ((), ())),
                                   preferred_element_type=jnp.float32))
    t = C_CONST * (am - CUT)
    ar_ref[0] = jnp.maximum(1.0 + t, 0.0) - jnp.maximum(t, 0.0)


def kernel(A, x, trafo):
    b, n, p = x.shape
    og = n // 2
    po = trafo.shape[1]
    values = x[:, :, -1]
    vrow = values.reshape(b, 1, n)
    vcol = values.reshape(b, n, 1)

    ar, traf = pl.pallas_call(
        _fused_body,
        grid=(b,),
        in_specs=[
            pl.BlockSpec((1, 1, n), lambda i: (i, 0, 0)),
            pl.BlockSpec((1, n, 1), lambda i: (i, 0, 0)),
            pl.BlockSpec((1, n, n), lambda i: (i, 0, 0)),
            pl.BlockSpec((1, n, p), lambda i: (i, 0, 0)),
            pl.BlockSpec((2 * p, po), lambda i: (0, 0)),
        ],
        out_specs=[
            pl.BlockSpec((1, og, og), lambda i: (i, 0, 0)),
            pl.BlockSpec((1, og, po), lambda i: (i, 0, 0)),
        ],
        out_shape=[
            jax.ShapeDtypeStruct((b, og, og), jnp.float32),
            jax.ShapeDtypeStruct((b, og, po), jnp.float32),
        ],
    )(vrow, vcol, A, x, trafo)

    return ar, traf
